# baseline (speedup 1.0000x reference)
import jax
import jax.numpy as jnp
from jax.experimental import pallas as pl
from jax.experimental.pallas import tpu as pltpu

_WD = 32
_NE = 64
_HC = 112
_PK = 4
_COST = 1.0 + 0.25


def _vq_body(x_ref, lA_ref, lB_ref, l2A_ref, l2B_ref, esq_ref,
             out_ref, loss_ref):
    j = pl.program_id(1)
    lA = lA_ref[...]
    lB = lB_ref[...]
    e_sq = esq_ref[...][:, 0:1]
    xb = x_ref[0]
    w = 224

    parts = []
    outs = []
    for g in range(_HC // 8):
        xg = xb[:, g * 8:(g + 1) * 8, :]
        xs = xg.reshape(_WD * 8, w)
        xsq8 = jnp.sum(xg * xg, axis=0)
        ohs = []
        for half, l in ((0, lA), (1, lB)):
            prod = jax.lax.dot_general(
                l, xs, (((1,), (0,)), ((), ())),
                preferred_element_type=jnp.float32,
                precision=jax.lax.Precision.DEFAULT)
            xsqh = xsq8[half * _PK:(half + 1) * _PK][:, None, :]
            d = ((xsqh + e_sq.reshape(_PK, _NE, 1))
                 + prod.reshape(_PK, _NE, w))
            iota = jax.lax.broadcasted_iota(jnp.int32, (_PK, _NE, w), 1)
            dv, iv = d, iota
            for level, width in enumerate((_NE, _NE // 2, _NE // 4)):
                half_w = width // 2
                av, bv = dv[:, :half_w], dv[:, half_w:width]
                ai, bi = iv[:, :half_w], iv[:, half_w:width]
                take_b = bv < av
                if level > 0:
                    take_b = jnp.logical_or(
                        take_b, jnp.logical_and(bv == av, bi < ai))
                dv = jnp.where(take_b, bv, av)
                iv = jnp.where(take_b, bi, ai)
            m = jnp.min(dv, axis=1, keepdims=True)
            sel = jnp.min(jnp.where(dv == m, iv, _NE), axis=1, keepdims=True)
            ohs.append((iota == sel).astype(jnp.float32).reshape(_PK * _NE, w))
            parts.append(jnp.sum(m))
        q = (jax.lax.dot_general(
                l2A_ref[...], ohs[0], (((1,), (0,)), ((), ())),
                preferred_element_type=jnp.float32,
                precision=jax.lax.Precision.DEFAULT)
             + jax.lax.dot_general(
                l2B_ref[...], ohs[1], (((1,), (0,)), ((), ())),
                preferred_element_type=jnp.float32,
                precision=jax.lax.Precision.DEFAULT))
        outs.append(q.reshape(_WD, 8, w))

    out_ref[0] = jnp.concatenate(outs, axis=1)
    part = sum(parts)

    @pl.when(j == 0)
    def _init():
        loss_ref[...] = jnp.zeros_like(loss_ref)

    loss_ref[...] += jnp.full(loss_ref.shape, part, jnp.float32)


def _mk_lhs(emb, h_offsets, scale):
    blocks = []
    for h in h_offsets:
        t = jnp.zeros((_NE, _WD, 8), emb.dtype).at[:, :, h].set(scale * emb)
        blocks.append(t.reshape(_NE, _WD * 8))
    return jnp.concatenate(blocks, axis=0)


def kernel(x, emb):
    b, c, h, w = x.shape
    lA = _mk_lhs(emb, [0, 1, 2, 3], -2.0)
    lB = _mk_lhs(emb, [4, 5, 6, 7], -2.0)
    l2A = _mk_lhs(emb, [0, 1, 2, 3], 1.0).T
    l2B = _mk_lhs(emb, [4, 5, 6, 7], 1.0).T
    e_sq = jnp.broadcast_to(
        jnp.tile(jnp.sum(emb * emb, axis=1), 4)[:, None], (256, 128))
    grid = (b, h // _HC)
    out, partials = pl.pallas_call(
        _vq_body,
        grid=grid,
        in_specs=[
            pl.BlockSpec((1, c, _HC, w), lambda i, j: (i, 0, j, 0)),
            pl.BlockSpec((256, 256), lambda i, j: (0, 0)),
            pl.BlockSpec((256, 256), lambda i, j: (0, 0)),
            pl.BlockSpec((256, 256), lambda i, j: (0, 0)),
            pl.BlockSpec((256, 256), lambda i, j: (0, 0)),
            pl.BlockSpec((256, 128), lambda i, j: (0, 0)),
        ],
        out_specs=[
            pl.BlockSpec((1, c, _HC, w), lambda i, j: (i, 0, j, 0)),
            pl.BlockSpec((1, 1, 128), lambda i, j: (i, 0, 0)),
        ],
        out_shape=[
            jax.ShapeDtypeStruct((b, c, h, w), jnp.float32),
            jax.ShapeDtypeStruct((b, 1, 128), jnp.float32),
        ],
        compiler_params=pltpu.CompilerParams(
            dimension_semantics=("parallel", "arbitrary")),
    )(x, lA, lB, l2A, l2B, e_sq)
    loss = (jnp.sum(partials[:, 0, 0]) * (_COST / x.size)).astype(jnp.float32)
    return (loss, out, 1, emb)


# speedup vs baseline: 1.0176x; 1.0176x over previous
import jax
import jax.numpy as jnp
from jax.experimental import pallas as pl
from jax.experimental.pallas import tpu as pltpu

_WD = 32
_NE = 64
_HC = 112
_PK = 4
_COST = 1.0 + 0.25


def _vq_body(x_ref, lA_ref, lB_ref, l2A_ref, l2B_ref, esq_ref,
             out_ref, loss_ref):
    j = pl.program_id(1)
    lA = lA_ref[...]
    lB = lB_ref[...]
    e_sq = esq_ref[...][:, 0:1]
    xb = x_ref[0]
    w = 224

    parts = []
    outs = []
    for g in range(_HC // 8):
        xg = xb[:, g * 8:(g + 1) * 8, :]
        xs = xg.reshape(_WD * 8, w)
        xs_bf = xs.astype(jnp.bfloat16)
        xsq8 = jnp.sum(xg * xg, axis=0)
        ohs = []
        for half, l in ((0, lA), (1, lB)):
            prod = jax.lax.dot_general(
                l, xs_bf, (((1,), (0,)), ((), ())),
                preferred_element_type=jnp.float32,
                precision=jax.lax.Precision.DEFAULT)
            xsqh = xsq8[half * _PK:(half + 1) * _PK][:, None, :]
            d = ((xsqh + e_sq.reshape(_PK, _NE, 1))
                 + prod.reshape(_PK, _NE, w))
            iota = jax.lax.broadcasted_iota(jnp.int32, (_PK, _NE, w), 1)
            dv, iv = d, iota
            for level, width in enumerate((_NE, _NE // 2, _NE // 4)):
                half_w = width // 2
                av, bv = dv[:, :half_w], dv[:, half_w:width]
                ai, bi = iv[:, :half_w], iv[:, half_w:width]
                take_b = bv < av
                if level > 0:
                    take_b = jnp.logical_or(
                        take_b, jnp.logical_and(bv == av, bi < ai))
                dv = jnp.where(take_b, bv, av)
                iv = jnp.where(take_b, bi, ai)
            m = jnp.min(dv, axis=1, keepdims=True)
            sel = jnp.min(jnp.where(dv == m, iv, _NE), axis=1, keepdims=True)
            ohs.append((iota == sel).astype(jnp.bfloat16).reshape(_PK * _NE, w))
            parts.append(jnp.sum(m))
        q = (jax.lax.dot_general(
                l2A_ref[...], ohs[0], (((1,), (0,)), ((), ())),
                preferred_element_type=jnp.float32,
                precision=jax.lax.Precision.DEFAULT)
             + jax.lax.dot_general(
                l2B_ref[...], ohs[1], (((1,), (0,)), ((), ())),
                preferred_element_type=jnp.float32,
                precision=jax.lax.Precision.DEFAULT))
        outs.append(q.reshape(_WD, 8, w))

    out_ref[0] = jnp.concatenate(outs, axis=1)
    part = sum(parts)

    @pl.when(j == 0)
    def _init():
        loss_ref[...] = jnp.zeros_like(loss_ref)

    loss_ref[...] += jnp.full(loss_ref.shape, part, jnp.float32)


def _mk_lhs(emb, h_offsets, scale):
    blocks = []
    for h in h_offsets:
        t = jnp.zeros((_NE, _WD, 8), emb.dtype).at[:, :, h].set(scale * emb)
        blocks.append(t.reshape(_NE, _WD * 8))
    return jnp.concatenate(blocks, axis=0)


def kernel(x, emb):
    b, c, h, w = x.shape
    lA = _mk_lhs(emb, [0, 1, 2, 3], -2.0).astype(jnp.bfloat16)
    lB = _mk_lhs(emb, [4, 5, 6, 7], -2.0).astype(jnp.bfloat16)
    l2A = _mk_lhs(emb, [0, 1, 2, 3], 1.0).T.astype(jnp.bfloat16)
    l2B = _mk_lhs(emb, [4, 5, 6, 7], 1.0).T.astype(jnp.bfloat16)
    e_sq = jnp.broadcast_to(
        jnp.tile(jnp.sum(emb * emb, axis=1), 4)[:, None], (256, 128))
    grid = (b, h // _HC)
    out, partials = pl.pallas_call(
        _vq_body,
        grid=grid,
        in_specs=[
            pl.BlockSpec((1, c, _HC, w), lambda i, j: (i, 0, j, 0)),
            pl.BlockSpec((256, 256), lambda i, j: (0, 0)),
            pl.BlockSpec((256, 256), lambda i, j: (0, 0)),
            pl.BlockSpec((256, 256), lambda i, j: (0, 0)),
            pl.BlockSpec((256, 256), lambda i, j: (0, 0)),
            pl.BlockSpec((256, 128), lambda i, j: (0, 0)),
        ],
        out_specs=[
            pl.BlockSpec((1, c, _HC, w), lambda i, j: (i, 0, j, 0)),
            pl.BlockSpec((1, 1, 128), lambda i, j: (i, 0, 0)),
        ],
        out_shape=[
            jax.ShapeDtypeStruct((b, c, h, w), jnp.float32),
            jax.ShapeDtypeStruct((b, 1, 128), jnp.float32),
        ],
        compiler_params=pltpu.CompilerParams(
            dimension_semantics=("parallel", "arbitrary")),
    )(x, lA, lB, l2A, l2B, e_sq)
    loss = (jnp.sum(partials[:, 0, 0]) * (_COST / x.size)).astype(jnp.float32)
    return (loss, out, 1, emb)


# speedup vs baseline: 1.0332x; 1.0153x over previous
import jax
import jax.numpy as jnp
from jax.experimental import pallas as pl
from jax.experimental.pallas import tpu as pltpu

_WD = 32
_NE = 64
_HC = 112
_PK = 4
_COST = 1.0 + 0.25


def _vq_body(x_ref, lA_ref, lB_ref, l2A_ref, l2B_ref, esq_ref,
             out_ref, loss_ref):
    j = pl.program_id(1)
    lA = lA_ref[...]
    lB = lB_ref[...]
    esqr = esq_ref[...].reshape(_PK, _NE, 224)
    xb = x_ref[0]
    w = 224

    parts = []
    for g in range(_HC // 8):
        xg = xb[:, g * 8:(g + 1) * 8, :]
        xs = xg.reshape(_WD * 8, w)
        xs_bf = xs.astype(jnp.bfloat16)
        xsq8 = jnp.sum(xg * xg, axis=0)
        ohs = []
        for half, l in ((0, lA), (1, lB)):
            prod = jax.lax.dot_general(
                l, xs_bf, (((1,), (0,)), ((), ())),
                preferred_element_type=jnp.float32,
                precision=jax.lax.Precision.DEFAULT)
            xsqh = xsq8[half * _PK:(half + 1) * _PK][:, None, :]
            d = (xsqh + esqr) + prod.reshape(_PK, _NE, w)
            iota = jax.lax.broadcasted_iota(jnp.int32, (_PK, _NE, w), 1)
            dv, iv = d, iota
            for level, width in enumerate((_NE, _NE // 2, _NE // 4)):
                half_w = width // 2
                av, bv = dv[:, :half_w], dv[:, half_w:width]
                ai, bi = iv[:, :half_w], iv[:, half_w:width]
                take_b = bv < av
                if level > 0:
                    take_b = jnp.logical_or(
                        take_b, jnp.logical_and(bv == av, bi < ai))
                dv = jnp.where(take_b, bv, av)
                iv = jnp.where(take_b, bi, ai)
            m = jnp.min(dv, axis=1, keepdims=True)
            sel = jnp.min(jnp.where(dv == m, iv, _NE), axis=1, keepdims=True)
            ohs.append((iota == sel).astype(jnp.bfloat16).reshape(_PK * _NE, w))
            parts.append(m)
        q = (jax.lax.dot_general(
                l2A_ref[...], ohs[0], (((1,), (0,)), ((), ())),
                preferred_element_type=jnp.float32,
                precision=jax.lax.Precision.DEFAULT)
             + jax.lax.dot_general(
                l2B_ref[...], ohs[1], (((1,), (0,)), ((), ())),
                preferred_element_type=jnp.float32,
                precision=jax.lax.Precision.DEFAULT))
        out_ref[0, :, g * 8:(g + 1) * 8, :] = q.reshape(_WD, 8, w)

    part = jnp.sum(sum(parts))

    @pl.when(j == 0)
    def _init():
        loss_ref[...] = jnp.zeros_like(loss_ref)

    loss_ref[...] += jnp.full(loss_ref.shape, part, jnp.float32)


def _mk_lhs(emb, h_offsets, scale):
    blocks = []
    for h in h_offsets:
        t = jnp.zeros((_NE, _WD, 8), emb.dtype).at[:, :, h].set(scale * emb)
        blocks.append(t.reshape(_NE, _WD * 8))
    return jnp.concatenate(blocks, axis=0)


def kernel(x, emb):
    b, c, h, w = x.shape
    lA = _mk_lhs(emb, [0, 1, 2, 3], -2.0).astype(jnp.bfloat16)
    lB = _mk_lhs(emb, [4, 5, 6, 7], -2.0).astype(jnp.bfloat16)
    l2A = _mk_lhs(emb, [0, 1, 2, 3], 1.0).T.astype(jnp.bfloat16)
    l2B = _mk_lhs(emb, [4, 5, 6, 7], 1.0).T.astype(jnp.bfloat16)
    e_sq = jnp.broadcast_to(
        jnp.tile(jnp.sum(emb * emb, axis=1), 4)[:, None], (256, 224))
    grid = (b, h // _HC)
    out, partials = pl.pallas_call(
        _vq_body,
        grid=grid,
        in_specs=[
            pl.BlockSpec((1, c, _HC, w), lambda i, j: (i, 0, j, 0)),
            pl.BlockSpec((256, 256), lambda i, j: (0, 0)),
            pl.BlockSpec((256, 256), lambda i, j: (0, 0)),
            pl.BlockSpec((256, 256), lambda i, j: (0, 0)),
            pl.BlockSpec((256, 256), lambda i, j: (0, 0)),
            pl.BlockSpec((256, 224), lambda i, j: (0, 0)),
        ],
        out_specs=[
            pl.BlockSpec((1, c, _HC, w), lambda i, j: (i, 0, j, 0)),
            pl.BlockSpec((1, 1, 128), lambda i, j: (i, 0, 0)),
        ],
        out_shape=[
            jax.ShapeDtypeStruct((b, c, h, w), jnp.float32),
            jax.ShapeDtypeStruct((b, 1, 128), jnp.float32),
        ],
        compiler_params=pltpu.CompilerParams(
            dimension_semantics=("parallel", "arbitrary")),
    )(x, lA, lB, l2A, l2B, e_sq)
    loss = (jnp.sum(partials[:, 0, 0]) * (_COST / x.size)).astype(jnp.float32)
    return (loss, out, 1, emb)


# speedup vs baseline: 1.0604x; 1.0263x over previous
import jax
import jax.numpy as jnp
from jax.experimental import pallas as pl
from jax.experimental.pallas import tpu as pltpu

_WD = 32
_NE = 64
_HC = 224
_PK = 4
_COST = 1.0 + 0.25


def _vq_body(x_ref, lA_ref, lB_ref, l2A_ref, l2B_ref, esq_ref,
             out_ref, loss_ref):
    j = pl.program_id(1)
    lA = lA_ref[...]
    lB = lB_ref[...]
    esqr = esq_ref[...].reshape(_PK, _NE, 224)
    xb = x_ref[0]
    w = 224

    parts = []
    for g in range(_HC // 8):
        xg = xb[:, g * 8:(g + 1) * 8, :]
        xs = xg.reshape(_WD * 8, w)
        xs_bf = xs.astype(jnp.bfloat16)
        xsq8 = jnp.sum(xg * xg, axis=0)
        ohs = []
        for half, l in ((0, lA), (1, lB)):
            prod = jax.lax.dot_general(
                l, xs_bf, (((1,), (0,)), ((), ())),
                preferred_element_type=jnp.float32,
                precision=jax.lax.Precision.DEFAULT)
            xsqh = xsq8[half * _PK:(half + 1) * _PK][:, None, :]
            d = (xsqh + esqr) + prod.reshape(_PK, _NE, w)
            iota = jax.lax.broadcasted_iota(jnp.int32, (_PK, _NE, w), 1)
            dv, iv = d, iota
            for level, width in enumerate((_NE, _NE // 2, _NE // 4)):
                half_w = width // 2
                av, bv = dv[:, :half_w], dv[:, half_w:width]
                ai, bi = iv[:, :half_w], iv[:, half_w:width]
                take_b = bv < av
                if level > 0:
                    take_b = jnp.logical_or(
                        take_b, jnp.logical_and(bv == av, bi < ai))
                dv = jnp.where(take_b, bv, av)
                iv = jnp.where(take_b, bi, ai)
            m = jnp.min(dv, axis=1, keepdims=True)
            sel = jnp.min(jnp.where(dv == m, iv, _NE), axis=1, keepdims=True)
            ohs.append((iota == sel).astype(jnp.bfloat16).reshape(_PK * _NE, w))
            parts.append(m)
        q = (jax.lax.dot_general(
                l2A_ref[...], ohs[0], (((1,), (0,)), ((), ())),
                preferred_element_type=jnp.float32,
                precision=jax.lax.Precision.DEFAULT)
             + jax.lax.dot_general(
                l2B_ref[...], ohs[1], (((1,), (0,)), ((), ())),
                preferred_element_type=jnp.float32,
                precision=jax.lax.Precision.DEFAULT))
        out_ref[0, :, g * 8:(g + 1) * 8, :] = q.reshape(_WD, 8, w)

    part = jnp.sum(sum(parts))

    @pl.when(j == 0)
    def _init():
        loss_ref[...] = jnp.zeros_like(loss_ref)

    loss_ref[...] += jnp.full(loss_ref.shape, part, jnp.float32)


def _mk_lhs(emb, h_offsets, scale):
    blocks = []
    for h in h_offsets:
        t = jnp.zeros((_NE, _WD, 8), emb.dtype).at[:, :, h].set(scale * emb)
        blocks.append(t.reshape(_NE, _WD * 8))
    return jnp.concatenate(blocks, axis=0)


def kernel(x, emb):
    b, c, h, w = x.shape
    lA = _mk_lhs(emb, [0, 1, 2, 3], -2.0).astype(jnp.bfloat16)
    lB = _mk_lhs(emb, [4, 5, 6, 7], -2.0).astype(jnp.bfloat16)
    l2A = _mk_lhs(emb, [0, 1, 2, 3], 1.0).T.astype(jnp.bfloat16)
    l2B = _mk_lhs(emb, [4, 5, 6, 7], 1.0).T.astype(jnp.bfloat16)
    e_sq = jnp.broadcast_to(
        jnp.tile(jnp.sum(emb * emb, axis=1), 4)[:, None], (256, 224))
    grid = (b, h // _HC)
    out, partials = pl.pallas_call(
        _vq_body,
        grid=grid,
        in_specs=[
            pl.BlockSpec((1, c, _HC, w), lambda i, j: (i, 0, j, 0)),
            pl.BlockSpec((256, 256), lambda i, j: (0, 0)),
            pl.BlockSpec((256, 256), lambda i, j: (0, 0)),
            pl.BlockSpec((256, 256), lambda i, j: (0, 0)),
            pl.BlockSpec((256, 256), lambda i, j: (0, 0)),
            pl.BlockSpec((256, 224), lambda i, j: (0, 0)),
        ],
        out_specs=[
            pl.BlockSpec((1, c, _HC, w), lambda i, j: (i, 0, j, 0)),
            pl.BlockSpec((1, 1, 128), lambda i, j: (i, 0, 0)),
        ],
        out_shape=[
            jax.ShapeDtypeStruct((b, c, h, w), jnp.float32),
            jax.ShapeDtypeStruct((b, 1, 128), jnp.float32),
        ],
        compiler_params=pltpu.CompilerParams(
            dimension_semantics=("parallel", "arbitrary")),
    )(x, lA, lB, l2A, l2B, e_sq)
    loss = (jnp.sum(partials[:, 0, 0]) * (_COST / x.size)).astype(jnp.float32)
    return (loss, out, 1, emb)
